# Initial kernel scaffold; baseline (speedup 1.0000x reference)
import jax
import jax.numpy as jnp
from jax.experimental import pallas as pl


def kernel(left_x, right_x, ucn_x, left_edge_weight, right_edge_weight, ucn_edge_weight, params, left_edge_index, right_edge_index, ucn_edge_index):
    raise NotImplementedError("write your pallas kernel here")


# speedup vs baseline: 4.1654x; 4.1654x over previous
import functools

import numpy as np
import jax
import jax.numpy as jnp
from jax import lax
from jax.experimental import pallas as pl
from jax.experimental.pallas import tpu as pltpu
from jax.experimental.pallas import tpu_sc as plsc

_B = 64
_NPH = 90
_D = 256
_H = 256
_OUT = 2
_NLAYER = 8
_NL = _B * _NPH
_N = 2 * _NL
_E = 184320
_ALPHA = 0.1
_THETA = 0.5
_LAMBA = 0.2
_ENTA = 0.7

_HW = 128
_NS = 16
_CH = 128
_EPS = _E // _NS
_NCH = _EPS // _CH
_RPS = _N // _NS
_G = 10
_NG = _NCH // _G

_BLK = 720


def _agg_sc_body(x0_hbm, x1_hbm, src_hbm, dst_hbm, w_hbm, zero_hbm,
                 out0_hbm, out1_hbm, src_v, dst_v, w_v, rows_v, acc, sem):
    c = lax.axis_index("c")
    s = lax.axis_index("s")

    pltpu.sync_copy(zero_hbm.at[pl.ds(s * _RPS, _RPS)],
                    acc.at[pl.ds(s * _RPS, _RPS)])
    plsc.subcore_barrier()

    def group(g, carry):
        pltpu.sync_copy(src_hbm.at[s, g], src_v)
        pltpu.sync_copy(dst_hbm.at[s, g], dst_v)
        pltpu.sync_copy(w_hbm.at[s, g], w_v)

        def chunk(j, carry2):
            @pl.when(c == 0)
            def _():
                pltpu.async_copy(x0_hbm.at[src_v.at[j]], rows_v, sem).wait()

            @pl.when(c == 1)
            def _():
                pltpu.async_copy(x1_hbm.at[src_v.at[j]], rows_v, sem).wait()

            def ebody(t, cc):
                w16 = w_v[j, pl.ds(t * 16, 16)]
                for k in range(16):
                    wv = jnp.full((16,), w16[k], jnp.float32)
                    e = t * 16 + k
                    for q in range(8):
                        sl = pl.ds(q * 16, 16)
                        rows_v[e, sl] = rows_v[e, sl] * wv
                return cc

            lax.fori_loop(0, _CH // 16, ebody, 0)
            pltpu.sync_copy(rows_v, acc.at[dst_v.at[j]], add=True)
            return carry2

        lax.fori_loop(0, _G, chunk, 0)
        return carry

    lax.fori_loop(0, _NG, group, 0)
    plsc.subcore_barrier()

    @pl.when(c == 0)
    def _():
        pltpu.sync_copy(acc.at[pl.ds(s * _RPS, _RPS)],
                        out0_hbm.at[pl.ds(s * _RPS, _RPS)])

    @pl.when(c == 1)
    def _():
        pltpu.sync_copy(acc.at[pl.ds(s * _RPS, _RPS)],
                        out1_hbm.at[pl.ds(s * _RPS, _RPS)])


@functools.cache
def _make_agg_sc():
    return pl.kernel(
        _agg_sc_body,
        out_type=[jax.ShapeDtypeStruct((_N, _HW), jnp.float32),
                  jax.ShapeDtypeStruct((_N, _HW), jnp.float32)],
        mesh=plsc.VectorSubcoreMesh(core_axis_name="c", subcore_axis_name="s"),
        scratch_types=[
            pltpu.VMEM((_G, _CH), jnp.int32),
            pltpu.VMEM((_G, _CH), jnp.int32),
            pltpu.VMEM((_G, _CH), jnp.float32),
            pltpu.VMEM((_CH, _HW), jnp.float32),
            pltpu.VMEM_SHARED((_N, _HW), jnp.float32),
            pltpu.SemaphoreType.DMA,
        ],
        name="edge_agg_sc",
    )


def _agg_sc(x0, x1, src, dst, w, zero):
    return _make_agg_sc()(x0, x1, src, dst, w, zero)


def _lin1_body(x_ref, w_ref, b_ref, o0_ref, o1_ref):
    y = jnp.dot(x_ref[...], w_ref[0], preferred_element_type=jnp.float32)
    y = y + b_ref[0]
    o0_ref[...] = y[:, :_HW]
    o1_ref[...] = y[:, _HW:]


def _lin1(xin, wstk, bstk):
    return pl.pallas_call(
        _lin1_body,
        grid=(_N // _BLK,),
        in_specs=[
            pl.BlockSpec((_BLK, _D), lambda i: (i, 0)),
            pl.BlockSpec((1, _D, _H), lambda i: (i // 8, 0, 0)),
            pl.BlockSpec((1, 1, _H), lambda i: (i // 8, 0, 0)),
        ],
        out_specs=[pl.BlockSpec((_BLK, _HW), lambda i: (i, 0)),
                   pl.BlockSpec((_BLK, _HW), lambda i: (i, 0))],
        out_shape=[jax.ShapeDtypeStruct((_N, _HW), jnp.float32),
                   jax.ShapeDtypeStruct((_N, _HW), jnp.float32)],
    )(xin, wstk, bstk)


def _combine_body(beta, h0, h1, x00, x01, wc_ref, g_ref, o0, o1, pool_ref):
    m0 = (1.0 - _ALPHA) * h0[...] + _ALPHA * x00[...]
    m1 = (1.0 - _ALPHA) * h1[...] + _ALPHA * x01[...]
    wc = wc_ref[0]
    mm = (jnp.dot(m0, wc[:_HW, :], preferred_element_type=jnp.float32)
          + jnp.dot(m1, wc[_HW:, :], preferred_element_type=jnp.float32))
    m = jnp.concatenate([m0, m1], axis=1)
    xn = (1.0 - beta) * m + beta * mm
    o0[...] = xn[:, :_HW]
    o1[...] = xn[:, _HW:]
    pool_ref[...] = jnp.dot(g_ref[...], xn, preferred_element_type=jnp.float32)


def _combine(beta, h0, h1, x00, x01, wc, gblk):
    return pl.pallas_call(
        functools.partial(_combine_body, beta),
        grid=(_N // _BLK,),
        in_specs=[
            pl.BlockSpec((_BLK, _HW), lambda i: (i, 0)),
            pl.BlockSpec((_BLK, _HW), lambda i: (i, 0)),
            pl.BlockSpec((_BLK, _HW), lambda i: (i, 0)),
            pl.BlockSpec((_BLK, _HW), lambda i: (i, 0)),
            pl.BlockSpec((1, _H, _H), lambda i: (i // 8, 0, 0)),
            pl.BlockSpec((8, _BLK), lambda i: (0, 0)),
        ],
        out_specs=[pl.BlockSpec((_BLK, _HW), lambda i: (i, 0)),
                   pl.BlockSpec((_BLK, _HW), lambda i: (i, 0)),
                   pl.BlockSpec((8, _H), lambda i: (i, 0))],
        out_shape=[jax.ShapeDtypeStruct((_N, _HW), jnp.float32),
                   jax.ShapeDtypeStruct((_N, _HW), jnp.float32),
                   jax.ShapeDtypeStruct((2 * _B, _H), jnp.float32)],
    )(h0, h1, x00, x01, wc, gblk)


def _gin1_body(h0, h1, x0p, x1p, w_ref, b_ref, eps_ref, o0, o1):
    a0 = h0[...] + x0p[...] * eps_ref[...]
    a1 = h1[...] + x1p[...] * eps_ref[...]
    y = (jnp.dot(a0, w_ref[:_HW, :], preferred_element_type=jnp.float32)
         + jnp.dot(a1, w_ref[_HW:, :], preferred_element_type=jnp.float32))
    y = y + b_ref[...]
    y = jax.nn.relu(y) * float(1.0 / np.sqrt(1.0 + 1e-5))
    o0[...] = y[:, :_HW]
    o1[...] = y[:, _HW:]


def _gin1(h0, h1, x0p, x1p, w, b, eps_row):
    return pl.pallas_call(
        _gin1_body,
        grid=(_N // _BLK,),
        in_specs=[
            pl.BlockSpec((_BLK, _HW), lambda i: (i, 0)),
            pl.BlockSpec((_BLK, _HW), lambda i: (i, 0)),
            pl.BlockSpec((_BLK, _HW), lambda i: (i, 0)),
            pl.BlockSpec((_BLK, _HW), lambda i: (i, 0)),
            pl.BlockSpec((_D, _H), lambda i: (0, 0)),
            pl.BlockSpec((1, _H), lambda i: (0, 0)),
            pl.BlockSpec((1, _HW), lambda i: (0, 0)),
        ],
        out_specs=[pl.BlockSpec((_BLK, _HW), lambda i: (i, 0)),
                   pl.BlockSpec((_BLK, _HW), lambda i: (i, 0))],
        out_shape=[jax.ShapeDtypeStruct((_N, _HW), jnp.float32),
                   jax.ShapeDtypeStruct((_N, _HW), jnp.float32)],
    )(h0, h1, x0p, x1p, w, b, eps_row)


def _gin2_body(h0, h1, x0p, x1p, w_ref, b_ref, eps_ref, g_ref, pool_ref):
    a0 = h0[...] + x0p[...] * eps_ref[...]
    a1 = h1[...] + x1p[...] * eps_ref[...]
    y = (jnp.dot(a0, w_ref[:_HW, :], preferred_element_type=jnp.float32)
         + jnp.dot(a1, w_ref[_HW:, :], preferred_element_type=jnp.float32))
    y = jax.nn.relu(y + b_ref[...])
    pool_ref[0] = jnp.dot(g_ref[...], y, preferred_element_type=jnp.float32)


def _gin2(h0, h1, x0p, x1p, w, b, eps_row, gblk4):
    return pl.pallas_call(
        _gin2_body,
        grid=(_N // _BLK,),
        in_specs=[
            pl.BlockSpec((_BLK, _HW), lambda i: (i, 0)),
            pl.BlockSpec((_BLK, _HW), lambda i: (i, 0)),
            pl.BlockSpec((_BLK, _HW), lambda i: (i, 0)),
            pl.BlockSpec((_BLK, _HW), lambda i: (i, 0)),
            pl.BlockSpec((_H, _H), lambda i: (0, 0)),
            pl.BlockSpec((1, _H), lambda i: (0, 0)),
            pl.BlockSpec((1, _HW), lambda i: (0, 0)),
            pl.BlockSpec((4, _BLK), lambda i: (0, 0)),
        ],
        out_specs=[pl.BlockSpec((1, 4, _H), lambda i: (i, 0, 0))],
        out_shape=[jax.ShapeDtypeStruct((_N // _BLK, 4, _H), jnp.float32)],
    )(h0, h1, x0p, x1p, w, b, eps_row, gblk4)


def _head_body(pool_ref, eu_ref, wfl, bfl, wfr, bfr, w2l, b2l, w2r, b2r,
               a1w, a1b, a2w, a2b, vgw, vgb, vcw, vcb, out_ref):
    def fuse(goff, wf, bf, w2, b2):
        acc = jnp.zeros((_B, _H), jnp.float32)
        for i in range(_NLAYER):
            acc = acc + jnp.dot(pool_ref[i, goff:goff + _B, :],
                                wf[i * _H:(i + 1) * _H, :],
                                preferred_element_type=jnp.float32)
        acc = acc + bf[...]
        return jnp.dot(acc, w2[...], preferred_element_type=jnp.float32) + b2[...]

    el = fuse(0, wfl, bfl, w2l, b2l)
    er = fuse(_B, wfr, bfr, w2r, b2r)
    eu = eu_ref[...]

    def nrm(z):
        return z / (jnp.sqrt(jnp.sum(z * z, axis=1, keepdims=True)) + 1e-6)

    znl, znr, znu = nrm(el), nrm(er), nrm(eu)

    def score(zn):
        h = jax.nn.relu(jnp.dot(zn, a1w[...],
                                preferred_element_type=jnp.float32) + a1b[...])
        return (jnp.dot(h, a2w[...], preferred_element_type=jnp.float32)
                + a2b[...]) * float(1.0 / np.sqrt(_H))

    s_l, s_r, s_u = score(znl), score(znr), score(znu)
    mx = jnp.maximum(jnp.maximum(s_l, s_r), s_u)
    xl, xr, xu = jnp.exp(s_l - mx), jnp.exp(s_r - mx), jnp.exp(s_u - mx)
    eg = (xl * znl + xr * znr + xu * znu) / (xl + xr + xu)

    def gate(e):
        return jax.nn.sigmoid(jnp.dot(e, vgw[...],
                                      preferred_element_type=jnp.float32)
                              + vgb[...])

    gl, gr, gu = gate(el), gate(er), gate(eu)
    local = (gl * el + gr * er + gu * eu) / (gl + gr + gu + 1e-6)
    fused = (_LAMBA * eg
             + (1.0 - _LAMBA) * (_ENTA * local
                                 + (1.0 - _ENTA) * (el + er + eu) / 3.0))
    logits = jnp.dot(fused, vcw[...], preferred_element_type=jnp.float32) + vcb[...]
    m = jnp.max(logits, axis=1, keepdims=True)
    ex = jnp.exp(logits - m)
    out_ref[...] = (logits - m) - jnp.log(jnp.sum(ex, axis=1, keepdims=True))


def _head(pool, eu, args):
    return pl.pallas_call(
        _head_body,
        out_shape=jax.ShapeDtypeStruct((_B, _OUT), jnp.float32),
    )(pool, eu, *args)


def kernel(left_x, right_x, ucn_x, left_edge_weight, right_edge_weight,
           ucn_edge_weight, params, left_edge_index, right_edge_index,
           ucn_edge_index):
    p = params
    f32 = jnp.float32

    srcg = jnp.concatenate([left_edge_index[0].astype(jnp.int32),
                            right_edge_index[0].astype(jnp.int32) + _NL])
    dstg = jnp.concatenate([left_edge_index[1].astype(jnp.int32),
                            right_edge_index[1].astype(jnp.int32) + _NL])
    wg = jnp.concatenate([left_edge_weight, right_edge_weight]).astype(f32)
    srcg = srcg.reshape(_NS, _NG, _G, _CH)
    dstg = dstg.reshape(_NS, _NG, _G, _CH)
    wg = wg.reshape(_NS, _NG, _G, _CH)

    srcu = ucn_edge_index[0].astype(jnp.int32).reshape(_NS, _NG, _G, _CH)
    dstu = ucn_edge_index[1].astype(jnp.int32).reshape(_NS, _NG, _G, _CH)
    wu = ucn_edge_weight.astype(f32).reshape(_NS, _NG, _G, _CH)

    zero = jnp.zeros((_N, _HW), f32)

    ids8 = np.repeat(np.arange(8), _NPH)
    gblk = jnp.asarray(
        (ids8[:, None] == np.arange(8)[None, :]).T.astype(np.float32) / _NPH)
    ids4 = np.repeat(np.arange(4), 2 * _NPH)
    gblk4 = jnp.asarray(
        (ids4[:, None] == np.arange(4)[None, :]).T.astype(np.float32)
        / (2 * _NPH))

    xin = jnp.concatenate([left_x, right_x], axis=0)
    wstk = jnp.stack([p['L_lin1']['w'], p['R_lin1']['w']])
    bstk = jnp.stack([p['L_lin1']['b'], p['R_lin1']['b']]).reshape(2, 1, _H)
    x0a, x0b = _lin1(xin, wstk, bstk)

    xa, xb = x0a, x0b
    pooled = []
    for i in range(_NLAYER):
        ha, hb = _agg_sc(xa, xb, srcg, dstg, wg, zero)
        beta = float(np.log(_THETA / (i + 1) + 1.0))
        wc = jnp.stack([p['L_conv'][i], p['R_conv'][i]])
        xa, xb, pool_i = _combine(beta, ha, hb, x0a, x0b, wc, gblk)
        pooled.append(pool_i)
    pool = jnp.stack(pooled)

    xu0 = ucn_x[:, :_HW]
    xu1 = ucn_x[:, _HW:]
    hu0, hu1 = _agg_sc(xu0, xu1, srcu, dstu, wu, zero)
    eps1_row = jnp.broadcast_to(1.0 + p['eps1'], (1, _HW)).astype(f32)
    eps2_row = jnp.broadcast_to(1.0 + p['eps2'], (1, _HW)).astype(f32)
    h10, h11 = _gin1(hu0, hu1, xu0, xu1, p['g1']['w'],
                     p['g1']['b'].reshape(1, _H), eps1_row)
    hu0, hu1 = _agg_sc(h10, h11, srcu, dstu, wu, zero)
    eu = _gin2(hu0, hu1, h10, h11, p['g2']['w'],
               p['g2']['b'].reshape(1, _H), eps2_row, gblk4)[0].reshape(_B, _H)

    args = (p['L_fus']['w'], p['L_fus']['b'].reshape(1, _H),
            p['R_fus']['w'], p['R_fus']['b'].reshape(1, _H),
            p['L_lin2']['w'], p['L_lin2']['b'].reshape(1, _H),
            p['R_lin2']['w'], p['R_lin2']['b'].reshape(1, _H),
            p['a1']['w'], p['a1']['b'].reshape(1, 16),
            p['a2']['w'], p['a2']['b'].reshape(1, 1),
            p['vg']['w'], p['vg']['b'].reshape(1, 1),
            p['vc']['w'], p['vc']['b'].reshape(1, _OUT))
    return _head(pool, eu, args)
